# baseline (speedup 1.0000x reference)
import functools

import jax
import jax.numpy as jnp
from jax import lax
from jax.experimental import pallas as pl
from jax.experimental.pallas import tpu as pltpu
from jax.experimental.pallas import tpu_sc as plsc

N = 50000
E = 800000
D = 64
Q = 16
BN = 1000
GRID = N // BN
EP = 819200
CH = 400
NCHUNK = 128
TEDGE = CH * NCHUNK
TRASH = N
RACC = 50016
ZCH = RACC // 16
WB = N // 16
EPS = 1e-5


def _sc_body(xa_hbm, xb_hbm, idx_hbm, z_hbm,
             aa_hbm, ab_hbm,
             ib0, ib1, rows0, rows1,
             acc, tbl, is0, is1, gs0, gs1, ss0, ss1):
    s = lax.axis_index("s")
    NROW = 16 * NCHUNK * 2

    def idx_slice(j):
        return pl.ds(jnp.minimum((s * NCHUNK + j) * 2, NROW - 2), 2)

    def issue_idx(j, ib, sem):
        pltpu.async_copy(idx_hbm.at[idx_slice(j)], ib, sem)

    def wait_idx(j, ib, sem):
        pltpu.make_async_copy(idx_hbm.at[idx_slice(j)], ib, sem).wait()

    def one_pass(table_hbm, out):
        zslc = pl.ds(s * ZCH, ZCH)
        pltpu.sync_copy(z_hbm.at[zslc], acc.at[zslc])
        tslc = pl.ds(s * WB, WB)
        pltpu.sync_copy(table_hbm.at[tslc], tbl.at[tslc])
        plsc.subcore_barrier()

        issue_idx(0, ib0, is0)
        issue_idx(1, ib1, is1)

        def pair(k, carry):
            j0 = 2 * k
            j1 = 2 * k + 1
            wait_idx(j0, ib0, is0)
            g0 = pltpu.async_copy(tbl.at[ib0.at[0]], rows0, gs0)
            wait_idx(j1, ib1, is1)
            g0.wait()
            sc0 = pltpu.async_copy(rows0, acc.at[ib0.at[1]], ss0, add=True)
            g1 = pltpu.async_copy(tbl.at[ib1.at[0]], rows1, gs1)
            sc0.wait()
            issue_idx(j0 + 2, ib0, is0)
            g1.wait()
            sc1 = pltpu.async_copy(rows1, acc.at[ib1.at[1]], ss1, add=True)
            sc1.wait()
            issue_idx(j1 + 2, ib1, is1)
            return carry
        lax.fori_loop(0, NCHUNK // 2, pair, 0)
        wait_idx(NCHUNK, ib0, is0)
        wait_idx(NCHUNK + 1, ib1, is1)

        plsc.subcore_barrier()
        oslc = pl.ds(s * WB, WB)
        pltpu.sync_copy(acc.at[oslc], out.at[oslc])
        plsc.subcore_barrier()

    one_pass(xa_hbm, aa_hbm)
    one_pass(xb_hbm, ab_hbm)


@functools.cache
def _get_sc_msgpass():
    qshape = jax.ShapeDtypeStruct((N, Q), jnp.float32)
    return functools.partial(
        pl.kernel,
        mesh=plsc.VectorSubcoreMesh(core_axis_name="c", subcore_axis_name="s",
                                    num_cores=1),
        compiler_params=pltpu.CompilerParams(use_tc_tiling_on_sc=False),
        out_type=(qshape, qshape),
        scratch_types=[
            pltpu.VMEM((2, CH), jnp.int32),
            pltpu.VMEM((2, CH), jnp.int32),
            pltpu.VMEM((CH, Q), jnp.float32),
            pltpu.VMEM((CH, Q), jnp.float32),
            pltpu.VMEM_SHARED((RACC, Q), jnp.float32),
            pltpu.VMEM_SHARED((N, Q), jnp.float32),
            pltpu.SemaphoreType.DMA,
            pltpu.SemaphoreType.DMA,
            pltpu.SemaphoreType.DMA,
            pltpu.SemaphoreType.DMA,
            pltpu.SemaphoreType.DMA,
            pltpu.SemaphoreType.DMA,
        ],
    )(_sc_body)


def _store_quarters(t, orefs):
    for q, o in enumerate(orefs):
        o[...] = t[:, q * Q:(q + 1) * Q]


def _cat_quarters(xrefs, arefs):
    return jnp.concatenate(
        [x[...] + a[...] for x, a in zip(xrefs, arefs)], axis=-1)


def _ln(h, g, b):
    mu = jnp.mean(h, axis=-1, keepdims=True)
    d = h - mu
    var = jnp.mean(d * d, axis=-1, keepdims=True)
    return d * lax.rsqrt(var + EPS) * g + b


def _mm1_body(x_ref, w_ref, b_ref, o0, o1, o2, o3):
    t = jnp.dot(x_ref[...], w_ref[...],
                preferred_element_type=jnp.float32) + b_ref[...]
    _store_quarters(t, (o0, o1, o2, o3))


def _fuse_body(x0, x1, x2, x3, a0, a1, a2, a3, g, bb, w, b, o0, o1, o2, o3):
    h = jnp.maximum(_cat_quarters((x0, x1, x2, x3), (a0, a1, a2, a3)), 0.0)
    y = _ln(h, g[...], bb[...])
    t = jnp.dot(y, w[...], preferred_element_type=jnp.float32) + b[...]
    _store_quarters(t, (o0, o1, o2, o3))


def _final_body(x0, x1, x2, x3, a0, a1, a2, a3, g, bb, og, ob, o):
    h = jnp.maximum(_cat_quarters((x0, x1, x2, x3), (a0, a1, a2, a3)), 0.0)
    y = _ln(h, g[...], bb[...])
    o[...] = _ln(y, og[...], ob[...])


_full = lambda i: (0, 0)
_rowblk_q = pl.BlockSpec((BN, Q), lambda i: (i, 0))
_qshape = jax.ShapeDtypeStruct((N, Q), jnp.float32)
_param = pl.BlockSpec((1, D), _full)

_mm1 = pl.pallas_call(
    _mm1_body,
    grid=(GRID,),
    in_specs=[
        pl.BlockSpec((BN, D), lambda i: (i % 10, 0)),
        pl.BlockSpec((D, D), _full),
        _param,
    ],
    out_specs=[_rowblk_q] * 4,
    out_shape=(_qshape,) * 4,
)

_fuse = pl.pallas_call(
    _fuse_body,
    grid=(GRID,),
    in_specs=[_rowblk_q] * 8 + [_param, _param, pl.BlockSpec((D, D), _full), _param],
    out_specs=[_rowblk_q] * 4,
    out_shape=(_qshape,) * 4,
)

_final = pl.pallas_call(
    _final_body,
    grid=(GRID,),
    in_specs=[_rowblk_q] * 8 + [_param] * 4,
    out_specs=pl.BlockSpec((BN, D), lambda i: (i, 0)),
    out_shape=jax.ShapeDtypeStruct((N, D), jnp.float32),
)


def kernel(edge_index, emb, W1, b1, W2, b2, W3, b3, ln_g, ln_b, out_g, out_b):
    row = edge_index[0]
    col = edge_index[1]
    colp = jnp.concatenate([col, jnp.zeros((EP - E,), jnp.int32)])
    rowp = jnp.concatenate([row, jnp.full((EP - E,), TRASH, jnp.int32)])
    idxp = jnp.stack(
        [colp.reshape(16 * NCHUNK, CH), rowp.reshape(16 * NCHUNK, CH)],
        axis=1).reshape(16 * NCHUNK * 2, CH)
    zacc = jnp.zeros((RACC, Q), jnp.float32)
    b1r, b2r, b3r = b1.reshape(1, D), b2.reshape(1, D), b3.reshape(1, D)
    gr, br = ln_g.reshape(1, D), ln_b.reshape(1, D)
    ogr, obr = out_g.reshape(1, D), out_b.reshape(1, D)

    sc_msgpass = _get_sc_msgpass()
    xs = _mm1(emb, W1, b1r)
    a01 = sc_msgpass(xs[0], xs[1], idxp, zacc)
    a23 = sc_msgpass(xs[2], xs[3], idxp, zacc)
    aa = (*a01, *a23)
    xs = _fuse(*xs, *aa, gr, br, W2, b2r)
    a01 = sc_msgpass(xs[0], xs[1], idxp, zacc)
    a23 = sc_msgpass(xs[2], xs[3], idxp, zacc)
    aa = (*a01, *a23)
    xs = _fuse(*xs, *aa, gr, br, W3, b3r)
    a01 = sc_msgpass(xs[0], xs[1], idxp, zacc)
    a23 = sc_msgpass(xs[2], xs[3], idxp, zacc)
    aa = (*a01, *a23)
    return _final(*xs, *aa, gr, br, ogr, obr)


# speedup vs baseline: 1.7559x; 1.7559x over previous
import functools

import jax
import jax.numpy as jnp
from jax import lax
from jax.experimental import pallas as pl
from jax.experimental.pallas import tpu as pltpu
from jax.experimental.pallas import tpu_sc as plsc

N = 50000
E = 800000
D = 64
Q = 16
N2 = 51200
NP = N2 // 2
BN = 512
GRID = NP // BN
EP = 819200
CH = 400
NCHUNK = 128
TEDGE = CH * NCHUNK
TRASH = N
RACC = 50016
ZCH = RACC // 16
WB = N // 16
EPS = 1e-5


def _sc_body(x_hbm, idx_hbm, z_hbm, ag_hbm,
             ib0, ib1, rows0, rows1,
             acc, tbl, is0, is1, gs0, gs1, ss0, ss1):
    c = lax.axis_index("c")
    s = lax.axis_index("s")
    NROW = 16 * NCHUNK * 2

    def idx_slice(j):
        return pl.ds(jnp.minimum((s * NCHUNK + j) * 2, NROW - 2), 2)

    def issue_idx(j, ib, sem):
        pltpu.async_copy(idx_hbm.at[idx_slice(j)], ib, sem)

    def wait_idx(j, ib, sem):
        pltpu.make_async_copy(idx_hbm.at[idx_slice(j)], ib, sem).wait()

    def one_pass(q):
        zslc = pl.ds(s * ZCH, ZCH)
        pltpu.sync_copy(z_hbm.at[zslc], acc.at[zslc])
        tslc = pl.ds(s * WB, WB)
        qslc = pl.ds(q * Q, Q)
        pltpu.sync_copy(x_hbm.at[tslc, qslc], tbl.at[tslc])
        plsc.subcore_barrier()

        issue_idx(0, ib0, is0)
        issue_idx(1, ib1, is1)

        def pair(k, carry):
            j0 = 2 * k
            j1 = 2 * k + 1
            wait_idx(j0, ib0, is0)
            g0 = pltpu.async_copy(tbl.at[ib0.at[0]], rows0, gs0)
            wait_idx(j1, ib1, is1)
            g0.wait()
            sc0 = pltpu.async_copy(rows0, acc.at[ib0.at[1]], ss0, add=True)
            g1 = pltpu.async_copy(tbl.at[ib1.at[0]], rows1, gs1)
            sc0.wait()
            issue_idx(j0 + 2, ib0, is0)
            g1.wait()
            sc1 = pltpu.async_copy(rows1, acc.at[ib1.at[1]], ss1, add=True)
            sc1.wait()
            issue_idx(j1 + 2, ib1, is1)
            return carry
        lax.fori_loop(0, NCHUNK // 2, pair, 0)
        wait_idx(NCHUNK, ib0, is0)
        wait_idx(NCHUNK + 1, ib1, is1)

        plsc.subcore_barrier()
        pltpu.sync_copy(acc.at[tslc], ag_hbm.at[tslc, qslc])
        plsc.subcore_barrier()

    @pl.when(c == 0)
    def _():
        one_pass(0)
        one_pass(1)

    @pl.when(c == 1)
    def _():
        one_pass(2)
        one_pass(3)


@functools.cache
def _get_sc_msgpass():
    return functools.partial(
        pl.kernel,
        mesh=plsc.VectorSubcoreMesh(core_axis_name="c", subcore_axis_name="s"),
        compiler_params=pltpu.CompilerParams(use_tc_tiling_on_sc=False),
        out_type=jax.ShapeDtypeStruct((N2, D), jnp.float32),
        scratch_types=[
            pltpu.VMEM((2, CH), jnp.int32),
            pltpu.VMEM((2, CH), jnp.int32),
            pltpu.VMEM((CH, Q), jnp.float32),
            pltpu.VMEM((CH, Q), jnp.float32),
            pltpu.VMEM_SHARED((RACC, Q), jnp.float32),
            pltpu.VMEM_SHARED((N, Q), jnp.float32),
            pltpu.SemaphoreType.DMA,
            pltpu.SemaphoreType.DMA,
            pltpu.SemaphoreType.DMA,
            pltpu.SemaphoreType.DMA,
            pltpu.SemaphoreType.DMA,
            pltpu.SemaphoreType.DMA,
        ],
    )(_sc_body)


def _ln_packed(h, hm, hmt, g2, b2):
    mu = jnp.dot(h, hm, preferred_element_type=jnp.float32) * (1.0 / D)
    ex2 = jnp.dot(h * h, hm, preferred_element_type=jnp.float32) * (1.0 / D)
    mub = jnp.dot(mu, hmt, preferred_element_type=jnp.float32)
    varb = jnp.dot(ex2 - mu * mu, hmt, preferred_element_type=jnp.float32)
    return (h - mub) * lax.rsqrt(varb + EPS) * g2 + b2


def _mm1_body(x_ref, w_ref, b_ref, o_ref):
    o_ref[...] = jnp.dot(x_ref[...], w_ref[...],
                         preferred_element_type=jnp.float32) + b_ref[...]


def _fuse_body(x_ref, a_ref, hm_ref, hmt_ref, g_ref, bb_ref, w_ref, b_ref,
               o_ref):
    h = jnp.maximum(x_ref[...] + a_ref[...], 0.0)
    y = _ln_packed(h, hm_ref[...], hmt_ref[...], g_ref[...], bb_ref[...])
    o_ref[...] = jnp.dot(y, w_ref[...],
                         preferred_element_type=jnp.float32) + b_ref[...]


def _final_body(x_ref, a_ref, hm_ref, hmt_ref, g_ref, bb_ref, og_ref, ob_ref,
                o_ref):
    h = jnp.maximum(x_ref[...] + a_ref[...], 0.0)
    y = _ln_packed(h, hm_ref[...], hmt_ref[...], g_ref[...], bb_ref[...])
    o_ref[...] = _ln_packed(y, hm_ref[...], hmt_ref[...], og_ref[...],
                            ob_ref[...])


_full = lambda i: (0, 0)
_blk = pl.BlockSpec((BN, 128), lambda i: (i, 0))
_pshape = jax.ShapeDtypeStruct((NP, 128), jnp.float32)
_wspec = pl.BlockSpec((128, 128), _full)
_vspec = pl.BlockSpec((1, 128), _full)
_hmspec = pl.BlockSpec((128, 2), _full)
_hmtspec = pl.BlockSpec((2, 128), _full)

_mm1 = pl.pallas_call(
    _mm1_body,
    grid=(GRID,),
    in_specs=[_blk, _wspec, _vspec],
    out_specs=_blk,
    out_shape=_pshape,
)

_fuse = pl.pallas_call(
    _fuse_body,
    grid=(GRID,),
    in_specs=[_blk, _blk, _hmspec, _hmtspec, _vspec, _vspec, _wspec, _vspec],
    out_specs=_blk,
    out_shape=_pshape,
)

_final = pl.pallas_call(
    _final_body,
    grid=(GRID,),
    in_specs=[_blk, _blk, _hmspec, _hmtspec, _vspec, _vspec, _vspec, _vspec],
    out_specs=_blk,
    out_shape=_pshape,
)


def _blockdiag2(W):
    z = jnp.zeros((D, D), jnp.float32)
    return jnp.block([[W, z], [z, W]])


def _tile2(v):
    return jnp.concatenate([v, v]).reshape(1, 128)


def kernel(edge_index, emb, W1, b1, W2, b2, W3, b3, ln_g, ln_b, out_g, out_b):
    row = edge_index[0]
    col = edge_index[1]
    colp = jnp.concatenate([col, jnp.zeros((EP - E,), jnp.int32)])
    rowp = jnp.concatenate([row, jnp.full((EP - E,), TRASH, jnp.int32)])
    idxp = jnp.stack(
        [colp.reshape(16 * NCHUNK, CH), rowp.reshape(16 * NCHUNK, CH)],
        axis=1).reshape(16 * NCHUNK * 2, CH)
    zacc = jnp.zeros((RACC, Q), jnp.float32)

    w1d, w2d, w3d = _blockdiag2(W1), _blockdiag2(W2), _blockdiag2(W3)
    b1t, b2t, b3t = _tile2(b1), _tile2(b2), _tile2(b3)
    gt, bt = _tile2(ln_g), _tile2(ln_b)
    ogt, obt = _tile2(out_g), _tile2(out_b)
    ii = jnp.arange(128)
    hm = jnp.where((ii < D)[:, None] == (jnp.arange(2) == 0)[None, :],
                   1.0, 0.0).astype(jnp.float32)
    hmt = hm.T

    embp = emb.reshape(emb.shape[0] // 2, 128)
    x0p = jnp.concatenate([embp] * 5
                          + [jnp.zeros((NP - 5 * embp.shape[0], 128),
                                       jnp.float32)])

    sc_msgpass = _get_sc_msgpass()

    def msg(xp):
        ag = sc_msgpass(xp.reshape(N2, D), idxp, zacc)
        return ag.reshape(NP, 128)

    xp = _mm1(x0p, w1d, b1t)
    ap = msg(xp)
    xp = _fuse(xp, ap, hm, hmt, gt, bt, w2d, b2t)
    ap = msg(xp)
    xp = _fuse(xp, ap, hm, hmt, gt, bt, w3d, b3t)
    ap = msg(xp)
    out = _final(xp, ap, hm, hmt, gt, bt, ogt, obt)
    return out.reshape(N2, D)[:N]
